# baseline (speedup 1.0000x reference)
import jax
import jax.numpy as jnp
from jax import lax
from jax.experimental import pallas as pl

B, L, D = 16, 4096, 128
GB = 4
NS = 4
QL = L // NS


def _body(x0_ref, x1_ref, x2_ref, x3_ref, m_ref, o_ref):
    m = m_ref[...].astype(jnp.float32)
    s = None
    for k, xr in enumerate((x0_ref, x1_ref, x2_ref, x3_ref)):
        p = lax.dot_general(m[:, :, k * QL:(k + 1) * QL], xr[:, 0],
                            (((2,), (1,)), ((0,), (0,))),
                            preferred_element_type=jnp.float32)
        s = p if s is None else s + p
    o_ref[...] = s / jnp.sum(m, axis=2, keepdims=True)


@jax.jit
def kernel(inputs, mask):
    x4 = inputs.reshape(B, NS, QL, D)
    m3 = mask.reshape(B, 1, L)
    out = pl.pallas_call(
        _body,
        grid=(B // GB,),
        in_specs=[
            pl.BlockSpec((GB, 1, QL, D), lambda b, k=k: (b, k, 0, 0))
            for k in range(NS)
        ] + [pl.BlockSpec((GB, 1, L), lambda b: (b, 0, 0))],
        out_specs=pl.BlockSpec((GB, 1, D), lambda b: (b, 0, 0)),
        out_shape=jax.ShapeDtypeStruct((B, 1, D), jnp.float32),
    )(x4, x4, x4, x4, m3)
    return out.reshape(B, D)
